# baseline (speedup 1.0000x reference)
import functools

import jax
import jax.numpy as jnp
from jax import lax
from jax.experimental import pallas as pl
from jax.experimental.pallas import tpu as pltpu
from jax.experimental.pallas import tpu_sc as plsc

N_NODES = 10000
N_EDGES = 160000
D_HALF = 128

NUM_CORES = 2
NUM_SUBCORES = 16
E_PER_TILE = N_EDGES // NUM_SUBCORES
CHUNK = 80
N_CHUNKS = E_PER_TILE // CHUNK
N_PAD = 10240
ROWS_PER_TILE = N_PAD // NUM_SUBCORES
INIT_ROWS = 128


def _bcast_lane(vec, lane):
    return lax.gather(
        vec,
        jnp.full((16, 1), lane, jnp.int32),
        lax.GatherDimensionNumbers(
            offset_dims=(),
            collapsed_slice_dims=(0,),
            start_index_map=(0,),
        ),
        (1,),
        mode=lax.GatherScatterMode.PROMISE_IN_BOUNDS,
    )


def _spmm_kernel(xa, xb, eidx):
    mesh = plsc.VectorSubcoreMesh(core_axis_name="c", subcore_axis_name="s")

    @functools.partial(
        pl.kernel,
        mesh=mesh,
        out_type=(
            jax.ShapeDtypeStruct((N_PAD, D_HALF), jnp.float32),
            jax.ShapeDtypeStruct((N_PAD, D_HALF), jnp.float32),
        ),
        scratch_types=[
            pltpu.VMEM((3, 3, CHUNK), jnp.int32),
            pltpu.VMEM((2, CHUNK, D_HALF), jnp.float32),
            pltpu.VMEM_SHARED((N_PAD, D_HALF), jnp.float32),
            pltpu.SemaphoreType.DMA,
            pltpu.SemaphoreType.DMA,
            pltpu.SemaphoreType.DMA,
        ],
    )
    def k(xa_ref, xb_ref, eidx_ref, ha_ref, hb_ref,
          idxc, rowbuf, hi_sh, gsem, isem, ssem):
        c = lax.axis_index("c")
        s = lax.axis_index("s")

        zero16 = jnp.zeros((16,), jnp.float32)

        def zrow(r, carry):
            for j in range(D_HALF // 16):
                rowbuf[0, r, pl.ds(j * 16, 16)] = zero16
            return carry

        lax.fori_loop(0, CHUNK, zrow, 0)
        for i in range(ROWS_PER_TILE // CHUNK):
            base = s * ROWS_PER_TILE + i * CHUNK
            pltpu.sync_copy(rowbuf.at[0], hi_sh.at[pl.ds(base, CHUNK)])
        plsc.subcore_barrier()

        def accumulate(table_ref, out_ref):
            pltpu.sync_copy(eidx_ref.at[s, 0], idxc.at[0])
            pltpu.async_copy(eidx_ref.at[s, 1], idxc.at[1], isem)
            pltpu.async_copy(table_ref.at[idxc.at[0, 0]], rowbuf.at[0], gsem)

            def chunk(j, carry):
                jm2 = lax.rem(j, 2)
                jp1m2 = lax.rem(j + 1, 2)
                jm3 = lax.rem(j, 3)
                jp1m3 = lax.rem(j + 1, 3)
                jp2m3 = lax.rem(j + 2, 3)

                @pl.when(j < N_CHUNKS - 1)
                def _():
                    pltpu.make_async_copy(
                        eidx_ref.at[s, 0], idxc.at[jp1m3], isem).wait()

                @pl.when(j >= 1)
                def _():
                    pltpu.make_async_copy(
                        rowbuf.at[jp1m2],
                        hi_sh.at[pl.ds(0, CHUNK)], ssem).wait()

                @pl.when(j < N_CHUNKS - 1)
                def _():
                    pltpu.async_copy(table_ref.at[idxc.at[jp1m3, 0]],
                                     rowbuf.at[jp1m2], gsem)

                @pl.when(j < N_CHUNKS - 2)
                def _():
                    pltpu.async_copy(eidx_ref.at[s, j + 2],
                                     idxc.at[jp2m3], isem)

                pltpu.make_async_copy(
                    table_ref.at[pl.ds(0, CHUNK)], rowbuf.at[jm2], gsem).wait()

                def edge_group(g, carry2):
                    evt = lax.bitcast_convert_type(
                        idxc[jm3, 2, pl.ds(g * 16, 16)], jnp.float32)
                    for e16 in range(16):
                        ev16 = _bcast_lane(evt, e16)
                        e = g * 16 + e16
                        for jj in range(D_HALF // 16):
                            sl = pl.ds(jj * 16, 16)
                            rowbuf[jm2, e, sl] = rowbuf[jm2, e, sl] * ev16
                    return carry2

                lax.fori_loop(0, CHUNK // 16, edge_group, 0)

                pltpu.async_copy(rowbuf.at[jm2], hi_sh.at[idxc.at[jm3, 1]],
                                 ssem, add=True)
                return carry

            lax.fori_loop(0, N_CHUNKS, chunk, 0)
            pltpu.make_async_copy(
                rowbuf.at[0], hi_sh.at[pl.ds(0, CHUNK)], ssem).wait()
            plsc.subcore_barrier()

            base = s * ROWS_PER_TILE
            pltpu.sync_copy(hi_sh.at[pl.ds(base, ROWS_PER_TILE)],
                            out_ref.at[pl.ds(base, ROWS_PER_TILE)])

        @pl.when(c == 0)
        def _():
            accumulate(xa_ref, ha_ref)

        @pl.when(c == 1)
        def _():
            accumulate(xb_ref, hb_ref)

    return k(xa, xb, eidx)


def _mm_body(ha_ref, hb_ref, wa_ref, wb_ref, b_ref, o_ref):
    acc = jnp.dot(ha_ref[...], wa_ref[...], preferred_element_type=jnp.float32)
    acc = acc + jnp.dot(hb_ref[...], wb_ref[...],
                        preferred_element_type=jnp.float32)
    o_ref[...] = acc + b_ref[...]


def _matmul(ha, hb, wa, wb, bias2):
    n, d_out = N_NODES, wa.shape[1]
    blk = 1000
    return pl.pallas_call(
        _mm_body,
        grid=(n // blk,),
        in_specs=[
            pl.BlockSpec((blk, D_HALF), lambda i: (i, 0)),
            pl.BlockSpec((blk, D_HALF), lambda i: (i, 0)),
            pl.BlockSpec((D_HALF, d_out), lambda i: (0, 0)),
            pl.BlockSpec((D_HALF, d_out), lambda i: (0, 0)),
            pl.BlockSpec((1, d_out), lambda i: (0, 0)),
        ],
        out_specs=pl.BlockSpec((blk, d_out), lambda i: (i, 0)),
        out_shape=jax.ShapeDtypeStruct((n, d_out), jnp.float32),
    )(ha, hb, wa, wb, bias2)


def kernel(edge_index, edge_vals, input_feature, weight, bias):
    ei = edge_index.astype(jnp.int32)
    row3 = ei[0].reshape(NUM_SUBCORES, N_CHUNKS, CHUNK)
    col3 = ei[1].reshape(NUM_SUBCORES, N_CHUNKS, CHUNK)
    ev3 = lax.bitcast_convert_type(
        edge_vals.astype(jnp.float32), jnp.int32
    ).reshape(NUM_SUBCORES, N_CHUNKS, CHUNK)
    eidx = jnp.stack([col3, row3, ev3], axis=2)
    xa = input_feature[:, :D_HALF]
    xb = input_feature[:, D_HALF:]
    ha, hb = _spmm_kernel(xa, xb, eidx)
    return _matmul(ha, hb, weight[:D_HALF], weight[D_HALF:],
                   bias.reshape(1, -1))


# speedup vs baseline: 2.5411x; 2.5411x over previous
import functools

import jax
import jax.numpy as jnp
from jax import lax
from jax.experimental import pallas as pl
from jax.experimental.pallas import tpu as pltpu
from jax.experimental.pallas import tpu_sc as plsc

N_NODES = 10000
N_EDGES = 160000
D_HALF = 128

NUM_CORES = 2
NUM_SUBCORES = 16
E_PER_TILE = N_EDGES // NUM_SUBCORES
CHUNK = 80
N_CHUNKS = E_PER_TILE // CHUNK
N_PAD = 10240
ROWS_PER_TILE = N_PAD // NUM_SUBCORES
INIT_ROWS = 128


def _bcast_lane(vec, lane):
    return lax.gather(
        vec,
        jnp.full((16, 1), lane, jnp.int32),
        lax.GatherDimensionNumbers(
            offset_dims=(),
            collapsed_slice_dims=(0,),
            start_index_map=(0,),
        ),
        (1,),
        mode=lax.GatherScatterMode.PROMISE_IN_BOUNDS,
    )


def _spmm_kernel(xa, xb, eidx):
    mesh = plsc.VectorSubcoreMesh(core_axis_name="c", subcore_axis_name="s")

    @functools.partial(
        pl.kernel,
        mesh=mesh,
        out_type=(
            jax.ShapeDtypeStruct((N_PAD, D_HALF), jnp.float32),
            jax.ShapeDtypeStruct((N_PAD, D_HALF), jnp.float32),
        ),
        scratch_types=[
            pltpu.VMEM((3, 3, CHUNK), jnp.int32),
            pltpu.VMEM((2, CHUNK, D_HALF), jnp.float32),
            pltpu.VMEM_SHARED((N_PAD, D_HALF), jnp.float32),
            pltpu.SemaphoreType.DMA,
            pltpu.SemaphoreType.DMA,
            pltpu.SemaphoreType.DMA,
        ],
    )
    def k(xa_ref, xb_ref, eidx_ref, ha_ref, hb_ref,
          idxc, rowbuf, hi_sh, gsem, isem, ssem):
        c = lax.axis_index("c")
        s = lax.axis_index("s")

        zero16 = jnp.zeros((16,), jnp.float32)

        def zrow(r, carry):
            for j in range(D_HALF // 16):
                rowbuf[0, r, pl.ds(j * 16, 16)] = zero16
            return carry

        lax.fori_loop(0, CHUNK, zrow, 0)
        for i in range(ROWS_PER_TILE // CHUNK):
            base = s * ROWS_PER_TILE + i * CHUNK
            pltpu.sync_copy(rowbuf.at[0], hi_sh.at[pl.ds(base, CHUNK)])
        plsc.subcore_barrier()

        def accumulate(table_ref, out_ref):
            pltpu.sync_copy(eidx_ref.at[s, 0], idxc.at[0])
            pltpu.async_copy(eidx_ref.at[s, 1], idxc.at[1], isem)
            pltpu.async_copy(table_ref.at[idxc.at[0, 0]], rowbuf.at[0], gsem)

            def chunk(j, carry):
                jm2 = lax.rem(j, 2)
                jp1m2 = lax.rem(j + 1, 2)
                jm3 = lax.rem(j, 3)
                jp1m3 = lax.rem(j + 1, 3)
                jp2m3 = lax.rem(j + 2, 3)

                @pl.when(j < N_CHUNKS - 1)
                def _():
                    pltpu.make_async_copy(
                        eidx_ref.at[s, 0], idxc.at[jp1m3], isem).wait()

                @pl.when(j >= 1)
                def _():
                    pltpu.make_async_copy(
                        rowbuf.at[jp1m2],
                        hi_sh.at[pl.ds(0, CHUNK)], ssem).wait()

                @pl.when(j < N_CHUNKS - 1)
                def _():
                    pltpu.async_copy(table_ref.at[idxc.at[jp1m3, 0]],
                                     rowbuf.at[jp1m2], gsem)

                @pl.when(j < N_CHUNKS - 2)
                def _():
                    pltpu.async_copy(eidx_ref.at[s, j + 2],
                                     idxc.at[jp2m3], isem)

                pltpu.make_async_copy(
                    table_ref.at[pl.ds(0, CHUNK)], rowbuf.at[jm2], gsem).wait()

                @plsc.parallel_loop(0, CHUNK, 1, unroll=4)
                def _(e):
                    base = e & ~jnp.int32(15)
                    lane = e & jnp.int32(15)
                    evt = lax.bitcast_convert_type(
                        idxc[jm3, 2, pl.ds(base, 16)], jnp.float32)
                    ev16 = _bcast_lane(evt, lane)
                    vals = [rowbuf[jm2, e, pl.ds(jj * 16, 16)]
                            for jj in range(D_HALF // 16)]
                    vals = [v * ev16 for v in vals]
                    for jj in range(D_HALF // 16):
                        rowbuf[jm2, e, pl.ds(jj * 16, 16)] = vals[jj]

                pltpu.async_copy(rowbuf.at[jm2], hi_sh.at[idxc.at[jm3, 1]],
                                 ssem, add=True)
                return carry

            lax.fori_loop(0, N_CHUNKS, chunk, 0)
            pltpu.make_async_copy(
                rowbuf.at[0], hi_sh.at[pl.ds(0, CHUNK)], ssem).wait()
            plsc.subcore_barrier()

            base = s * ROWS_PER_TILE
            pltpu.sync_copy(hi_sh.at[pl.ds(base, ROWS_PER_TILE)],
                            out_ref.at[pl.ds(base, ROWS_PER_TILE)])

        @pl.when(c == 0)
        def _():
            accumulate(xa_ref, ha_ref)

        @pl.when(c == 1)
        def _():
            accumulate(xb_ref, hb_ref)

    return k(xa, xb, eidx)


def _mm_body(ha_ref, hb_ref, wa_ref, wb_ref, b_ref, o_ref):
    acc = jnp.dot(ha_ref[...], wa_ref[...], preferred_element_type=jnp.float32)
    acc = acc + jnp.dot(hb_ref[...], wb_ref[...],
                        preferred_element_type=jnp.float32)
    o_ref[...] = acc + b_ref[...]


def _matmul(ha, hb, wa, wb, bias2):
    n, d_out = N_NODES, wa.shape[1]
    blk = 1000
    return pl.pallas_call(
        _mm_body,
        grid=(n // blk,),
        in_specs=[
            pl.BlockSpec((blk, D_HALF), lambda i: (i, 0)),
            pl.BlockSpec((blk, D_HALF), lambda i: (i, 0)),
            pl.BlockSpec((D_HALF, d_out), lambda i: (0, 0)),
            pl.BlockSpec((D_HALF, d_out), lambda i: (0, 0)),
            pl.BlockSpec((1, d_out), lambda i: (0, 0)),
        ],
        out_specs=pl.BlockSpec((blk, d_out), lambda i: (i, 0)),
        out_shape=jax.ShapeDtypeStruct((n, d_out), jnp.float32),
    )(ha, hb, wa, wb, bias2)


def kernel(edge_index, edge_vals, input_feature, weight, bias):
    ei = edge_index.astype(jnp.int32)
    row3 = ei[0].reshape(NUM_SUBCORES, N_CHUNKS, CHUNK)
    col3 = ei[1].reshape(NUM_SUBCORES, N_CHUNKS, CHUNK)
    ev3 = lax.bitcast_convert_type(
        edge_vals.astype(jnp.float32), jnp.int32
    ).reshape(NUM_SUBCORES, N_CHUNKS, CHUNK)
    eidx = jnp.stack([col3, row3, ev3], axis=2)
    xa = input_feature[:, :D_HALF]
    xb = input_feature[:, D_HALF:]
    ha, hb = _spmm_kernel(xa, xb, eidx)
    return _matmul(ha, hb, weight[:D_HALF], weight[D_HALF:],
                   bias.reshape(1, -1))


# speedup vs baseline: 2.8267x; 1.1124x over previous
import functools

import jax
import jax.numpy as jnp
from jax import lax
from jax.experimental import pallas as pl
from jax.experimental.pallas import tpu as pltpu
from jax.experimental.pallas import tpu_sc as plsc

N_NODES = 10000
N_EDGES = 160000
D_HALF = 128

NUM_CORES = 2
NUM_SUBCORES = 16
E_PER_TILE = N_EDGES // NUM_SUBCORES
CHUNK = 80
N_CHUNKS = E_PER_TILE // CHUNK
N_PAD = 10240
ROWS_PER_TILE = N_PAD // NUM_SUBCORES
INIT_ROWS = 128


def _bcast_lane(vec, lane):
    return lax.gather(
        vec,
        jnp.full((16, 1), lane, jnp.int32),
        lax.GatherDimensionNumbers(
            offset_dims=(),
            collapsed_slice_dims=(0,),
            start_index_map=(0,),
        ),
        (1,),
        mode=lax.GatherScatterMode.PROMISE_IN_BOUNDS,
    )


def _spmm_kernel(xa, xb, eidx):
    mesh = plsc.VectorSubcoreMesh(core_axis_name="c", subcore_axis_name="s")

    @functools.partial(
        pl.kernel,
        mesh=mesh,
        out_type=(
            jax.ShapeDtypeStruct((N_PAD, D_HALF), jnp.float32),
            jax.ShapeDtypeStruct((N_PAD, D_HALF), jnp.float32),
        ),
        scratch_types=[
            pltpu.VMEM((5, 3, CHUNK), jnp.int32),
            pltpu.VMEM((3, CHUNK, D_HALF), jnp.float32),
            pltpu.VMEM_SHARED((N_PAD, D_HALF), jnp.float32),
            pltpu.SemaphoreType.DMA,
            pltpu.SemaphoreType.DMA,
            pltpu.SemaphoreType.DMA,
        ],
    )
    def k(xa_ref, xb_ref, eidx_ref, ha_ref, hb_ref,
          idxc, rowbuf, hi_sh, gsem, isem, ssem):
        c = lax.axis_index("c")
        s = lax.axis_index("s")

        zero16 = jnp.zeros((16,), jnp.float32)

        def zrow(r, carry):
            for j in range(D_HALF // 16):
                rowbuf[0, r, pl.ds(j * 16, 16)] = zero16
            return carry

        lax.fori_loop(0, CHUNK, zrow, 0)
        for i in range(ROWS_PER_TILE // CHUNK):
            base = s * ROWS_PER_TILE + i * CHUNK
            pltpu.sync_copy(rowbuf.at[0], hi_sh.at[pl.ds(base, CHUNK)])
        plsc.subcore_barrier()

        def accumulate(table_ref, out_ref):
            pltpu.sync_copy(eidx_ref.at[s, 0], idxc.at[0])
            pltpu.sync_copy(eidx_ref.at[s, 1], idxc.at[1])
            pltpu.async_copy(eidx_ref.at[s, 2], idxc.at[2], isem)
            pltpu.async_copy(eidx_ref.at[s, 3], idxc.at[3], isem)
            pltpu.async_copy(table_ref.at[idxc.at[0, 0]], rowbuf.at[0], gsem)
            pltpu.async_copy(table_ref.at[idxc.at[1, 0]], rowbuf.at[1], gsem)

            def chunk(j, carry):
                jm3 = lax.rem(j, 3)
                jp2m3 = lax.rem(j + 2, 3)
                jm5 = lax.rem(j, 5)
                jp2m5 = lax.rem(j + 2, 5)
                jp4m5 = lax.rem(j + 4, 5)

                @pl.when(j < N_CHUNKS - 2)
                def _():
                    pltpu.make_async_copy(
                        eidx_ref.at[s, 0], idxc.at[jp2m5], isem).wait()

                @pl.when(j >= 1)
                def _():
                    pltpu.make_async_copy(
                        rowbuf.at[jp2m3],
                        hi_sh.at[pl.ds(0, CHUNK)], ssem).wait()

                @pl.when(j < N_CHUNKS - 2)
                def _():
                    pltpu.async_copy(table_ref.at[idxc.at[jp2m5, 0]],
                                     rowbuf.at[jp2m3], gsem)

                @pl.when(j < N_CHUNKS - 4)
                def _():
                    pltpu.async_copy(eidx_ref.at[s, j + 4],
                                     idxc.at[jp4m5], isem)

                pltpu.make_async_copy(
                    table_ref.at[pl.ds(0, CHUNK)], rowbuf.at[jm3], gsem).wait()

                @plsc.parallel_loop(0, CHUNK, 1, unroll=4)
                def _(e):
                    base = e & ~jnp.int32(15)
                    lane = e & jnp.int32(15)
                    evt = lax.bitcast_convert_type(
                        idxc[jm5, 2, pl.ds(base, 16)], jnp.float32)
                    ev16 = _bcast_lane(evt, lane)
                    vals = [rowbuf[jm3, e, pl.ds(jj * 16, 16)]
                            for jj in range(D_HALF // 16)]
                    vals = [v * ev16 for v in vals]
                    for jj in range(D_HALF // 16):
                        rowbuf[jm3, e, pl.ds(jj * 16, 16)] = vals[jj]

                pltpu.async_copy(rowbuf.at[jm3], hi_sh.at[idxc.at[jm5, 1]],
                                 ssem, add=True)
                return carry

            lax.fori_loop(0, N_CHUNKS, chunk, 0)
            pltpu.make_async_copy(
                rowbuf.at[0], hi_sh.at[pl.ds(0, CHUNK)], ssem).wait()
            plsc.subcore_barrier()

            base = s * ROWS_PER_TILE
            pltpu.sync_copy(hi_sh.at[pl.ds(base, ROWS_PER_TILE)],
                            out_ref.at[pl.ds(base, ROWS_PER_TILE)])

        @pl.when(c == 0)
        def _():
            accumulate(xa_ref, ha_ref)

        @pl.when(c == 1)
        def _():
            accumulate(xb_ref, hb_ref)

    return k(xa, xb, eidx)


def _mm_body(ha_ref, hb_ref, wa_ref, wb_ref, b_ref, o_ref):
    acc = jnp.dot(ha_ref[...], wa_ref[...], preferred_element_type=jnp.float32)
    acc = acc + jnp.dot(hb_ref[...], wb_ref[...],
                        preferred_element_type=jnp.float32)
    o_ref[...] = acc + b_ref[...]


def _matmul(ha, hb, wa, wb, bias2):
    n, d_out = N_NODES, wa.shape[1]
    blk = 1000
    return pl.pallas_call(
        _mm_body,
        grid=(n // blk,),
        in_specs=[
            pl.BlockSpec((blk, D_HALF), lambda i: (i, 0)),
            pl.BlockSpec((blk, D_HALF), lambda i: (i, 0)),
            pl.BlockSpec((D_HALF, d_out), lambda i: (0, 0)),
            pl.BlockSpec((D_HALF, d_out), lambda i: (0, 0)),
            pl.BlockSpec((1, d_out), lambda i: (0, 0)),
        ],
        out_specs=pl.BlockSpec((blk, d_out), lambda i: (i, 0)),
        out_shape=jax.ShapeDtypeStruct((n, d_out), jnp.float32),
    )(ha, hb, wa, wb, bias2)


def kernel(edge_index, edge_vals, input_feature, weight, bias):
    ei = edge_index.astype(jnp.int32)
    row3 = ei[0].reshape(NUM_SUBCORES, N_CHUNKS, CHUNK)
    col3 = ei[1].reshape(NUM_SUBCORES, N_CHUNKS, CHUNK)
    ev3 = lax.bitcast_convert_type(
        edge_vals.astype(jnp.float32), jnp.int32
    ).reshape(NUM_SUBCORES, N_CHUNKS, CHUNK)
    eidx = jnp.stack([col3, row3, ev3], axis=2)
    xa = input_feature[:, :D_HALF]
    xb = input_feature[:, D_HALF:]
    ha, hb = _spmm_kernel(xa, xb, eidx)
    return _matmul(ha, hb, weight[:D_HALF], weight[D_HALF:],
                   bias.reshape(1, -1))
